# baseline (speedup 1.0000x reference)
import functools

import jax
import jax.numpy as jnp
from jax import lax
from jax.experimental import pallas as pl
from jax.experimental.pallas import tpu as pltpu
from jax.experimental.pallas import tpu_sc as plsc

ZCH_SIZE = 1_000_000
EMBED_DIM = 64
PAD_DIM = 128
N_IDS = 327_680
CHUNK = 128
ROWS = N_IDS // CHUNK
NW = 32
RPW = ROWS // NW
NBUF = 4
GROUPS = RPW // NBUF


def _sc_body(ids0, ids1, t0, t1, rem0, rem1, part,
             idx_v, gidx_v, off_v, rows_v, acc_v, s0, s1, s2, s3):
    cid = lax.axis_index("c")
    sid = lax.axis_index("s")
    wid = sid * 2 + cid
    sems = (s0, s1, s2, s3)

    def fire(tv, i, b):
        pltpu.async_copy(tv.at[gidx_v.at[i]], rows_v.at[b], sems[b])

    def drain(tv, b):
        pltpu.make_async_copy(tv.at[pl.ds(0, CHUNK)], rows_v.at[b], sems[b]).wait()

    def do_feature(ids_hbm, tv, rem_hbm, acc):
        base = wid * RPW
        pltpu.sync_copy(ids_hbm.at[pl.ds(base, RPW)], idx_v)

        def remap_body(i, c):
            for j in range(CHUNK // 16):
                x = idx_v[i, pl.ds(j * 16, 16)]
                x = x - jnp.where(x >= 2 * ZCH_SIZE,
                                  jnp.int32(2 * ZCH_SIZE), jnp.int32(0))
                x = x - jnp.where(x >= ZCH_SIZE,
                                  jnp.int32(ZCH_SIZE), jnp.int32(0))
                idx_v[i, pl.ds(j * 16, 16)] = x
                gidx_v[i, pl.ds(j * 16, 16)] = lax.shift_right_logical(x, 1)
                off_v[i, pl.ds(j * 16, 16)] = lax.shift_left(
                    lax.bitwise_and(x, 1), 6)
            return c

        lax.fori_loop(0, RPW, remap_body, 0)
        pltpu.sync_copy(idx_v, rem_hbm.at[pl.ds(base, RPW)])

        for b in range(NBUF):
            fire(tv, jnp.int32(b), b)

        def group_body(g, acc):
            for b in range(NBUF):
                drain(tv, b)
                acc = _accum_slab(rows_v.at[b], off_v, g * NBUF + b, acc)

                @pl.when(g < GROUPS - 1)
                def _():
                    fire(tv, (g + 1) * NBUF + b, b)

            return acc

        return lax.fori_loop(0, GROUPS, group_body, acc)

    z = jnp.zeros((16,), jnp.float32)
    acc = (z,) * 8
    acc = do_feature(ids0, t0, rem0, acc)
    acc = do_feature(ids1, t1, rem1, acc)
    tot = acc[0]
    for k in range(1, 8):
        tot = tot + acc[k]
    acc_v[pl.ds(0, 16)] = tot
    for k in range(1, 8):
        acc_v[pl.ds(k * 16, 16)] = z
    pltpu.sync_copy(acc_v, part.at[wid])


def _accum_slab(slab, off_v, chunk_i, acc):
    def group16(q, acc):
        a = list(acc)
        rvec = q * 16 + lax.iota(jnp.int32, 16)
        ovec = off_v[chunk_i, pl.ds(q * 16, 16)]
        for k in range(EMBED_DIM):
            g = plsc.load_gather(slab, [rvec, ovec + k])
            a[k % 8] = a[k % 8] + g
        return tuple(a)

    return lax.fori_loop(0, CHUNK // 16, group16, acc)


@jax.jit
def kernel(ids_0, ids_1, table_0, table_1):
    mesh = plsc.VectorSubcoreMesh(core_axis_name="c", subcore_axis_name="s")
    run = functools.partial(
        pl.kernel,
        out_type=(
            jax.ShapeDtypeStruct((ROWS, CHUNK), jnp.int32),
            jax.ShapeDtypeStruct((ROWS, CHUNK), jnp.int32),
            jax.ShapeDtypeStruct((NW, PAD_DIM), jnp.float32),
        ),
        mesh=mesh,
        scratch_types=(
            pltpu.VMEM((RPW, CHUNK), jnp.int32),
            pltpu.VMEM((RPW, CHUNK), jnp.int32),
            pltpu.VMEM((RPW, CHUNK), jnp.int32),
            pltpu.VMEM((NBUF, CHUNK, PAD_DIM), jnp.float32),
            pltpu.VMEM((PAD_DIM,), jnp.float32),
            pltpu.SemaphoreType.DMA,
            pltpu.SemaphoreType.DMA,
            pltpu.SemaphoreType.DMA,
            pltpu.SemaphoreType.DMA,
        ),
        compiler_params=pltpu.CompilerParams(needs_layout_passes=False),
    )(_sc_body)
    rem0, rem1, part = run(
        ids_0.reshape(ROWS, CHUNK),
        ids_1.reshape(ROWS, CHUNK),
        table_0.reshape(ZCH_SIZE // 2, PAD_DIM),
        table_1.reshape(ZCH_SIZE // 2, PAD_DIM),
    )
    loss = part.sum() / jnp.float32(2 * N_IDS * EMBED_DIM)
    return (loss, rem0.reshape(-1), rem1.reshape(-1))


# speedup vs baseline: 1.5893x; 1.5893x over previous
import functools

import jax
import jax.numpy as jnp
from jax import lax
from jax.experimental import pallas as pl
from jax.experimental.pallas import tpu as pltpu
from jax.experimental.pallas import tpu_sc as plsc

ZCH_SIZE = 1_000_000
EMBED_DIM = 64
N_IDS = 327_680
CHUNK = 128
NW = 32
IPW = N_IDS // NW
CPW = IPW // CHUNK
NBUF = 4
GROUPS = CPW // NBUF


def _sc_body(ids0, ids1, t0, t1, rem0, rem1, part,
             idx_v, rows_v, acc_v, s0, s1, s2, s3):
    cid = lax.axis_index("c")
    sid = lax.axis_index("s")
    wid = sid * 2 + cid
    sems = (s0, s1, s2, s3)

    def fire(t_hbm, i, b):
        pltpu.async_copy(t_hbm.at[idx_v.at[pl.ds(i * CHUNK, CHUNK)]],
                         rows_v.at[b], sems[b])

    def drain(t_hbm, b):
        pltpu.make_async_copy(t_hbm.at[pl.ds(0, CHUNK)], rows_v.at[b], sems[b]).wait()

    def do_feature(ids_hbm, t_hbm, rem_hbm, acc):
        base = wid * IPW
        pltpu.sync_copy(ids_hbm.at[pl.ds(base, IPW)], idx_v)

        def remap_body(i, c):
            for j in range(CHUNK // 16):
                o = i * CHUNK + j * 16
                x = idx_v[pl.ds(o, 16)]
                x = x - jnp.where(x >= 2 * ZCH_SIZE,
                                  jnp.int32(2 * ZCH_SIZE), jnp.int32(0))
                x = x - jnp.where(x >= ZCH_SIZE,
                                  jnp.int32(ZCH_SIZE), jnp.int32(0))
                idx_v[pl.ds(o, 16)] = x
            return c

        lax.fori_loop(0, CPW, remap_body, 0)
        pltpu.sync_copy(idx_v, rem_hbm.at[pl.ds(base, IPW)])

        for b in range(NBUF):
            fire(t_hbm, jnp.int32(b), b)

        def group_body(g, acc):
            for b in range(NBUF):
                drain(t_hbm, b)

                def row_body(r, acc):
                    a = list(acc)
                    for u in range(4):
                        rr = r * 4 + u
                        o = (u % 2) * 4
                        a[o + 0] = a[o + 0] + rows_v[b, rr, pl.ds(0, 16)]
                        a[o + 1] = a[o + 1] + rows_v[b, rr, pl.ds(16, 16)]
                        a[o + 2] = a[o + 2] + rows_v[b, rr, pl.ds(32, 16)]
                        a[o + 3] = a[o + 3] + rows_v[b, rr, pl.ds(48, 16)]
                    return tuple(a)

                acc = lax.fori_loop(0, CHUNK // 4, row_body, acc)

                @pl.when(g < GROUPS - 1)
                def _():
                    fire(t_hbm, (g + 1) * NBUF + b, b)

            return acc

        return lax.fori_loop(0, GROUPS, group_body, acc)

    z = jnp.zeros((16,), jnp.float32)
    acc = (z,) * 8
    acc = do_feature(ids0, t0, rem0, acc)
    acc = do_feature(ids1, t1, rem1, acc)
    tot = acc[0]
    for k in range(1, 8):
        tot = tot + acc[k]
    acc_v[...] = tot
    pltpu.sync_copy(acc_v, part.at[wid])


@jax.jit
def kernel(ids_0, ids_1, table_0, table_1):
    mesh = plsc.VectorSubcoreMesh(core_axis_name="c", subcore_axis_name="s")
    run = functools.partial(
        pl.kernel,
        out_type=(
            jax.ShapeDtypeStruct((N_IDS,), jnp.int32),
            jax.ShapeDtypeStruct((N_IDS,), jnp.int32),
            jax.ShapeDtypeStruct((NW, 16), jnp.float32),
        ),
        mesh=mesh,
        scratch_types=(
            pltpu.VMEM((IPW,), jnp.int32),
            pltpu.VMEM((NBUF, CHUNK, EMBED_DIM), jnp.float32),
            pltpu.VMEM((16,), jnp.float32),
            pltpu.SemaphoreType.DMA,
            pltpu.SemaphoreType.DMA,
            pltpu.SemaphoreType.DMA,
            pltpu.SemaphoreType.DMA,
        ),
        compiler_params=pltpu.CompilerParams(use_tc_tiling_on_sc=False),
    )(_sc_body)
    rem0, rem1, part = run(ids_0, ids_1, table_0, table_1)
    loss = part.sum() / jnp.float32(2 * N_IDS * EMBED_DIM)
    return (loss, rem0, rem1)


# speedup vs baseline: 1.5895x; 1.0001x over previous
import functools

import jax
import jax.numpy as jnp
from jax import lax
from jax.experimental import pallas as pl
from jax.experimental.pallas import tpu as pltpu
from jax.experimental.pallas import tpu_sc as plsc

ZCH_SIZE = 1_000_000
EMBED_DIM = 64
PAD_DIM = 128
N_IDS = 327_680
CHUNK = 128
NW = 32
IPW = N_IDS // NW
CPW = IPW // CHUNK
NBUF = 4
GROUPS = CPW // NBUF


def _sc_body(ids0, ids1, t0, t1, rem0, rem1, part,
             idx_v, rows_v, acc_v, s0, s1, s2, s3):
    cid = lax.axis_index("c")
    sid = lax.axis_index("s")
    wid = sid * 2 + cid
    sems = (s0, s1, s2, s3)

    def fire(t_hbm, i, b):
        pltpu.async_copy(t_hbm.at[idx_v.at[pl.ds(i * CHUNK, CHUNK)]],
                         rows_v.at[b], sems[b])

    def drain(t_hbm, b):
        pltpu.make_async_copy(t_hbm.at[pl.ds(0, CHUNK)], rows_v.at[b], sems[b]).wait()

    def do_feature(ids_hbm, t_hbm, rem_hbm, acc):
        base = wid * IPW
        pltpu.sync_copy(ids_hbm.at[pl.ds(base, IPW)], idx_v)

        def remap_body(i, c):
            for j in range(CHUNK // 16):
                o = i * CHUNK + j * 16
                x = idx_v[pl.ds(o, 16)]
                x = x - jnp.where(x >= 2 * ZCH_SIZE,
                                  jnp.int32(2 * ZCH_SIZE), jnp.int32(0))
                x = x - jnp.where(x >= ZCH_SIZE,
                                  jnp.int32(ZCH_SIZE), jnp.int32(0))
                idx_v[pl.ds(o, 16)] = x
            return c

        lax.fori_loop(0, CPW, remap_body, 0)
        pltpu.sync_copy(idx_v, rem_hbm.at[pl.ds(base, IPW)])

        for b in range(NBUF):
            fire(t_hbm, jnp.int32(b), b)

        def group_body(g, acc):
            for b in range(NBUF):
                drain(t_hbm, b)

                def row_body(r, acc):
                    a = list(acc)
                    for u in range(4):
                        rr = r * 4 + u
                        o = (u % 2) * 4
                        a[o + 0] = a[o + 0] + rows_v[b, rr, pl.ds(0, 16)]
                        a[o + 1] = a[o + 1] + rows_v[b, rr, pl.ds(16, 16)]
                        a[o + 2] = a[o + 2] + rows_v[b, rr, pl.ds(32, 16)]
                        a[o + 3] = a[o + 3] + rows_v[b, rr, pl.ds(48, 16)]
                    return tuple(a)

                acc = lax.fori_loop(0, CHUNK // 4, row_body, acc)

                @pl.when(g < GROUPS - 1)
                def _():
                    fire(t_hbm, (g + 1) * NBUF + b, b)

            return acc

        return lax.fori_loop(0, GROUPS, group_body, acc)

    z = jnp.zeros((16,), jnp.float32)
    acc = (z,) * 8
    acc = do_feature(ids0, t0, rem0, acc)
    acc = do_feature(ids1, t1, rem1, acc)
    tot = acc[0]
    for k in range(1, 8):
        tot = tot + acc[k]
    acc_v[...] = tot
    pltpu.sync_copy(acc_v, part.at[wid])


@jax.jit
def kernel(ids_0, ids_1, table_0, table_1):
    mesh = plsc.VectorSubcoreMesh(core_axis_name="c", subcore_axis_name="s")
    run = functools.partial(
        pl.kernel,
        out_type=(
            jax.ShapeDtypeStruct((N_IDS,), jnp.int32),
            jax.ShapeDtypeStruct((N_IDS,), jnp.int32),
            jax.ShapeDtypeStruct((NW, 16), jnp.float32),
        ),
        mesh=mesh,
        scratch_types=(
            pltpu.VMEM((IPW,), jnp.int32),
            pltpu.VMEM((NBUF, CHUNK, PAD_DIM), jnp.float32),
            pltpu.VMEM((16,), jnp.float32),
            pltpu.SemaphoreType.DMA,
            pltpu.SemaphoreType.DMA,
            pltpu.SemaphoreType.DMA,
            pltpu.SemaphoreType.DMA,
        ),
    )(_sc_body)
    tp0 = jnp.pad(table_0, ((0, 0), (0, PAD_DIM - EMBED_DIM)))
    tp1 = jnp.pad(table_1, ((0, 0), (0, PAD_DIM - EMBED_DIM)))
    rem0, rem1, part = run(ids_0, ids_1, tp0, tp1)
    loss = part.sum() / jnp.float32(2 * N_IDS * EMBED_DIM)
    return (loss, rem0, rem1)
